# Initial kernel scaffold; baseline (speedup 1.0000x reference)
import jax
import jax.numpy as jnp
from jax.experimental import pallas as pl


def kernel(x, edge_index, Wl, Wr, bc, gamma, beta, alpha, W1, b1, W2, b2, W3, b3):
    raise NotImplementedError("write your pallas kernel here")


# speedup vs baseline: 3.0842x; 3.0842x over previous
import functools

import jax
import jax.numpy as jnp
from jax import lax
from jax.experimental import pallas as pl
from jax.experimental.pallas import tpu as pltpu
from jax.experimental.pallas import tpu_sc as plsc

NN = 10000
EE = 320000
DD = 128
MM = 256
OUTS = (7, 2, 5, 4, 4)

NC = 2
NS = 16
NW = NC * NS
CH = 128
ROWS_PER_W = 80
R_PAD = NW * ROWS_PER_W
N_PAD = 10112
ROWS_PER_TILE = N_PAD // NS
IDX_CH = 16

_mesh = plsc.VectorSubcoreMesh(
    core_axis_name="c", subcore_axis_name="s", num_cores=NC, num_subcores=NS)


@functools.partial(
    pl.kernel,
    out_type=jax.ShapeDtypeStruct((NC * N_PAD, DD), jnp.float32),
    mesh=_mesh,
    scratch_types=[
        pltpu.VMEM((IDX_CH, CH), jnp.int32),
        pltpu.VMEM((IDX_CH, CH), jnp.int32),
        pltpu.VMEM((CH,), jnp.int32),
        pltpu.VMEM((CH, DD), jnp.float32),
        pltpu.VMEM((CH, DD), jnp.float32),
        pltpu.VMEM_SHARED((N_PAD, DD), jnp.float32),
        pltpu.SemaphoreType.DMA,
        pltpu.SemaphoreType.DMA,
    ],
)
def _sc_segsum(h_hbm, src_hbm, dst_hbm, z_hbm, out_hbm,
               src_v, dst_v, dst_cur, buf0, buf1, acc, sem0, sem1):
    cid = lax.axis_index("c")
    sid = lax.axis_index("s")
    wid = cid * NS + sid

    row0 = sid * ROWS_PER_TILE
    pltpu.sync_copy(z_hbm.at[pl.ds(row0, ROWS_PER_TILE)],
                    acc.at[pl.ds(row0, ROWS_PER_TILE)])
    plsc.subcore_barrier()

    def _stage(st, carry):
        base = wid * ROWS_PER_W + st * IDX_CH
        pltpu.sync_copy(src_hbm.at[pl.ds(base, IDX_CH)], src_v)
        pltpu.sync_copy(dst_hbm.at[pl.ds(base, IDX_CH)], dst_v)

        pltpu.async_copy(h_hbm.at[src_v.at[0]], buf0, sem0)
        pltpu.async_copy(h_hbm.at[src_v.at[1]], buf1, sem1)

        def _scat(buf, j):
            for g in range(CH // 16):
                dst_cur[pl.ds(g * 16, 16)] = dst_v[j, pl.ds(g * 16, 16)]
            pltpu.sync_copy(buf, acc.at[dst_cur], add=True)

        def _body(jj, c2):
            j0 = 2 * jj
            pltpu.make_async_copy(h_hbm.at[src_v.at[j0]], buf0, sem0).wait()
            _scat(buf0, j0)
            pltpu.async_copy(h_hbm.at[src_v.at[j0 + 2]], buf0, sem0)
            pltpu.make_async_copy(
                h_hbm.at[src_v.at[j0 + 1]], buf1, sem1).wait()
            _scat(buf1, j0 + 1)
            pltpu.async_copy(h_hbm.at[src_v.at[j0 + 3]], buf1, sem1)
            return c2
        lax.fori_loop(0, IDX_CH // 2 - 1, _body, 0)

        jlast = IDX_CH - 2
        pltpu.make_async_copy(h_hbm.at[src_v.at[jlast]], buf0, sem0).wait()
        _scat(buf0, jlast)
        pltpu.make_async_copy(
            h_hbm.at[src_v.at[jlast + 1]], buf1, sem1).wait()
        _scat(buf1, jlast + 1)
        return carry
    lax.fori_loop(0, ROWS_PER_W // IDX_CH, _stage, 0)

    plsc.subcore_barrier()
    pltpu.sync_copy(acc.at[pl.ds(row0, ROWS_PER_TILE)],
                    out_hbm.at[pl.ds(cid * N_PAD + row0, ROWS_PER_TILE)])


DEGW = 128


@functools.partial(
    pl.kernel,
    out_type=jax.ShapeDtypeStruct((NC * N_PAD, DEGW), jnp.float32),
    mesh=_mesh,
    scratch_types=[
        pltpu.VMEM((IDX_CH, CH), jnp.int32),
        pltpu.VMEM((CH,), jnp.int32),
        pltpu.VMEM((CH, DEGW), jnp.float32),
        pltpu.VMEM_SHARED((N_PAD, DEGW), jnp.float32),
    ],
)
def _sc_degree(dst_hbm, zd_hbm, ones_hbm, out_hbm, dst_v, dst_cur, buf, dacc):
    cid = lax.axis_index("c")
    sid = lax.axis_index("s")
    wid = cid * NS + sid

    row0 = sid * ROWS_PER_TILE
    pltpu.sync_copy(zd_hbm.at[pl.ds(row0, ROWS_PER_TILE)],
                    dacc.at[pl.ds(row0, ROWS_PER_TILE)])
    plsc.subcore_barrier()

    pltpu.sync_copy(ones_hbm, buf)

    def _stage(st, carry):
        base = wid * ROWS_PER_W + st * IDX_CH
        pltpu.sync_copy(dst_hbm.at[pl.ds(base, IDX_CH)], dst_v)

        def _body(j, c2):
            for g in range(CH // 16):
                dst_cur[pl.ds(g * 16, 16)] = dst_v[j, pl.ds(g * 16, 16)]
            pltpu.sync_copy(buf, dacc.at[dst_cur], add=True)
            return c2
        lax.fori_loop(0, IDX_CH, _body, 0)
        return carry
    lax.fori_loop(0, ROWS_PER_W // IDX_CH, _stage, 0)

    plsc.subcore_barrier()
    pltpu.sync_copy(dacc.at[pl.ds(row0, ROWS_PER_TILE)],
                    out_hbm.at[pl.ds(cid * N_PAD + row0, ROWS_PER_TILE)])


def _rdeg_body(parts_ref, out_ref):
    deg = (parts_ref[0:N_PAD, 0:1] + parts_ref[N_PAD:2 * N_PAD, 0:1])
    out_ref[...] = 1.0 / jnp.maximum(deg, 1.0)


_rdeg = pl.pallas_call(
    _rdeg_body,
    out_shape=jax.ShapeDtypeStruct((N_PAD, 1), jnp.float32),
)


def _layer_body(s0_ref, s1_ref, rdeg_ref, h_ref, wl_ref, wr_ref, b_ref,
                g_ref, be_ref, al_ref, out_ref):
    s = s0_ref[...] + s1_ref[...]
    mean = s[:NN] * rdeg_ref[...][:NN]
    t = (jnp.dot(mean, wl_ref[...], preferred_element_type=jnp.float32)
         + jnp.dot(h_ref[...], wr_ref[...], preferred_element_type=jnp.float32)
         + b_ref[...])
    mu = jnp.mean(t, axis=0, keepdims=True)
    o = t - al_ref[...] * mu
    var = jnp.mean(o * o, axis=0, keepdims=True)
    out_ref[...] = jnp.maximum(
        g_ref[...] * o * lax.rsqrt(var + 1e-5) + be_ref[...], 0.0)


_layer_tc = pl.pallas_call(
    _layer_body,
    out_shape=jax.ShapeDtypeStruct((NN, DD), jnp.float32),
)


_BR = 2000


def _heads_body(h_ref, w1_ref, b1_ref, w2_ref, b2_ref,
                w30, w31, w32, w33, w34, b30, b31, b32, b33, b34,
                o0, o1, o2, o3, o4):
    hv = h_ref[...]
    w3s = (w30, w31, w32, w33, w34)
    b3s = (b30, b31, b32, b33, b34)
    outs = (o0, o1, o2, o3, o4)
    for i in range(5):
        z = jnp.maximum(
            jnp.dot(hv, w1_ref[i], preferred_element_type=jnp.float32)
            + b1_ref[i], 0.0)
        z = jnp.maximum(
            jnp.dot(z, w2_ref[i], preferred_element_type=jnp.float32)
            + b2_ref[i], 0.0)
        outs[i][...] = (jnp.dot(z, w3s[i][...],
                                preferred_element_type=jnp.float32)
                        + b3s[i][...])


_heads = pl.pallas_call(
    _heads_body,
    grid=(NN // _BR,),
    in_specs=[
        pl.BlockSpec((_BR, DD), lambda i: (i, 0)),
        pl.BlockSpec((5, DD, MM), lambda i: (0, 0, 0)),
        pl.BlockSpec((5, MM), lambda i: (0, 0)),
        pl.BlockSpec((5, MM, MM // 2), lambda i: (0, 0, 0)),
        pl.BlockSpec((5, MM // 2), lambda i: (0, 0)),
    ] + [pl.BlockSpec((MM // 2, o), lambda i: (0, 0)) for o in OUTS]
      + [pl.BlockSpec((1, o), lambda i: (0, 0)) for o in OUTS],
    out_specs=[pl.BlockSpec((_BR, o), lambda i: (i, 0)) for o in OUTS],
    out_shape=[jax.ShapeDtypeStruct((NN, o), jnp.float32) for o in OUTS],
)


def kernel(x, edge_index, Wl, Wr, bc, gamma, beta, alpha,
           W1, b1, W2, b2, W3, b3):
    src = edge_index[0]
    dst = edge_index[1]
    pad = R_PAD * CH - EE
    src_p = jnp.concatenate(
        [src, jnp.zeros((pad,), jnp.int32)]).reshape(R_PAD, CH)
    dst_p = jnp.concatenate(
        [dst, jnp.full((pad,), NN, jnp.int32)]).reshape(R_PAD, CH)

    zeros_deg = jnp.zeros((N_PAD, DEGW), jnp.float32)
    ones_deg = jnp.ones((CH, DEGW), jnp.float32)
    zeros_feat = jnp.zeros((N_PAD, DD), jnp.float32)

    deg_parts = _sc_degree(dst_p, zeros_deg, ones_deg)
    rdeg = _rdeg(deg_parts)

    h = x
    for i in range(5):
        s2 = _sc_segsum(h, src_p, dst_p, zeros_feat).reshape(NC, N_PAD, DD)
        h = _layer_tc(s2[0], s2[1], rdeg, h, Wl[i], Wr[i],
                      bc[i][None], gamma[i][None], beta[i][None],
                      alpha[i][None])

    outs = _heads(h, W1, b1, W2, b2, *W3, *[b[None] for b in b3])
    return tuple(outs)


# speedup vs baseline: 3.0877x; 1.0011x over previous
import functools

import jax
import jax.numpy as jnp
from jax import lax
from jax.experimental import pallas as pl
from jax.experimental.pallas import tpu as pltpu
from jax.experimental.pallas import tpu_sc as plsc

NN = 10000
EE = 320000
DD = 128
MM = 256
OUTS = (7, 2, 5, 4, 4)

NC = 2
NS = 16
NW = NC * NS
CH = 128
ROWS_PER_W = 80
R_PAD = NW * ROWS_PER_W
N_PAD = 10112
ROWS_PER_TILE = N_PAD // NS
IDX_CH = 16

_mesh = plsc.VectorSubcoreMesh(
    core_axis_name="c", subcore_axis_name="s", num_cores=NC, num_subcores=NS)


@functools.partial(
    pl.kernel,
    out_type=jax.ShapeDtypeStruct((NC * N_PAD, DD), jnp.float32),
    mesh=_mesh,
    scratch_types=[
        pltpu.VMEM((IDX_CH, CH), jnp.int32),
        pltpu.VMEM((IDX_CH, CH), jnp.int32),
        pltpu.VMEM((CH,), jnp.int32),
        pltpu.VMEM((CH, DD), jnp.float32),
        pltpu.VMEM((CH, DD), jnp.float32),
        pltpu.VMEM_SHARED((N_PAD, DD), jnp.float32),
        pltpu.SemaphoreType.DMA,
        pltpu.SemaphoreType.DMA,
    ],
)
def _sc_segsum(h_hbm, src_hbm, dst_hbm, z_hbm, out_hbm,
               src_v, dst_v, dst_cur, buf0, buf1, acc, sem0, sem1):
    cid = lax.axis_index("c")
    sid = lax.axis_index("s")
    wid = cid * NS + sid

    row0 = sid * ROWS_PER_TILE
    pltpu.sync_copy(z_hbm.at[pl.ds(row0, ROWS_PER_TILE)],
                    acc.at[pl.ds(row0, ROWS_PER_TILE)])
    plsc.subcore_barrier()

    def _stage(st, carry):
        base = wid * ROWS_PER_W + st * IDX_CH
        pltpu.sync_copy(src_hbm.at[pl.ds(base, IDX_CH)], src_v)
        pltpu.sync_copy(dst_hbm.at[pl.ds(base, IDX_CH)], dst_v)

        pltpu.async_copy(h_hbm.at[src_v.at[0]], buf0, sem0)
        pltpu.async_copy(h_hbm.at[src_v.at[1]], buf1, sem1)

        def _scat(buf, j):
            for g in range(CH // 16):
                dst_cur[pl.ds(g * 16, 16)] = dst_v[j, pl.ds(g * 16, 16)]
            pltpu.sync_copy(buf, acc.at[dst_cur], add=True)

        def _body(jj, c2):
            j0 = 2 * jj
            pltpu.make_async_copy(h_hbm.at[src_v.at[j0]], buf0, sem0).wait()
            _scat(buf0, j0)
            pltpu.async_copy(h_hbm.at[src_v.at[j0 + 2]], buf0, sem0)
            pltpu.make_async_copy(
                h_hbm.at[src_v.at[j0 + 1]], buf1, sem1).wait()
            _scat(buf1, j0 + 1)
            pltpu.async_copy(h_hbm.at[src_v.at[j0 + 3]], buf1, sem1)
            return c2
        lax.fori_loop(0, IDX_CH // 2 - 1, _body, 0)

        jlast = IDX_CH - 2
        pltpu.make_async_copy(h_hbm.at[src_v.at[jlast]], buf0, sem0).wait()
        _scat(buf0, jlast)
        pltpu.make_async_copy(
            h_hbm.at[src_v.at[jlast + 1]], buf1, sem1).wait()
        _scat(buf1, jlast + 1)
        return carry
    lax.fori_loop(0, ROWS_PER_W // IDX_CH, _stage, 0)

    plsc.subcore_barrier()
    pltpu.sync_copy(acc.at[pl.ds(row0, ROWS_PER_TILE)],
                    out_hbm.at[pl.ds(cid * N_PAD + row0, ROWS_PER_TILE)])


DEGW = 128


@functools.partial(
    pl.kernel,
    out_type=jax.ShapeDtypeStruct((NC * N_PAD, DEGW), jnp.float32),
    mesh=_mesh,
    scratch_types=[
        pltpu.VMEM((IDX_CH, CH), jnp.int32),
        pltpu.VMEM((CH,), jnp.int32),
        pltpu.VMEM((CH, DEGW), jnp.float32),
        pltpu.VMEM_SHARED((N_PAD, DEGW), jnp.float32),
    ],
)
def _sc_degree(dst_hbm, zd_hbm, ones_hbm, out_hbm, dst_v, dst_cur, buf, dacc):
    cid = lax.axis_index("c")
    sid = lax.axis_index("s")
    wid = cid * NS + sid

    row0 = sid * ROWS_PER_TILE
    pltpu.sync_copy(zd_hbm.at[pl.ds(row0, ROWS_PER_TILE)],
                    dacc.at[pl.ds(row0, ROWS_PER_TILE)])
    plsc.subcore_barrier()

    pltpu.sync_copy(ones_hbm, buf)

    def _stage(st, carry):
        base = wid * ROWS_PER_W + st * IDX_CH
        pltpu.sync_copy(dst_hbm.at[pl.ds(base, IDX_CH)], dst_v)

        def _body(j, c2):
            for g in range(CH // 16):
                dst_cur[pl.ds(g * 16, 16)] = dst_v[j, pl.ds(g * 16, 16)]
            pltpu.sync_copy(buf, dacc.at[dst_cur], add=True)
            return c2
        lax.fori_loop(0, IDX_CH, _body, 0)
        return carry
    lax.fori_loop(0, ROWS_PER_W // IDX_CH, _stage, 0)

    plsc.subcore_barrier()
    pltpu.sync_copy(dacc.at[pl.ds(row0, ROWS_PER_TILE)],
                    out_hbm.at[pl.ds(cid * N_PAD + row0, ROWS_PER_TILE)])


def _rdeg_body(parts_ref, out_ref):
    deg = (parts_ref[0:N_PAD, 0:1] + parts_ref[N_PAD:2 * N_PAD, 0:1])
    out_ref[...] = 1.0 / jnp.maximum(deg, 1.0)


_rdeg = pl.pallas_call(
    _rdeg_body,
    out_shape=jax.ShapeDtypeStruct((N_PAD, 1), jnp.float32),
)


def _layer_body(s0_ref, s1_ref, rdeg_ref, h_ref, wl_ref, wr_ref, b_ref,
                g_ref, be_ref, al_ref, out_ref):
    s = s0_ref[...] + s1_ref[...]
    mean = s[:NN] * rdeg_ref[...][:NN]
    t = (jnp.dot(mean, wl_ref[...], preferred_element_type=jnp.float32)
         + jnp.dot(h_ref[...], wr_ref[...], preferred_element_type=jnp.float32)
         + b_ref[...])
    mu = jnp.mean(t, axis=0, keepdims=True)
    o = t - al_ref[...] * mu
    var = jnp.mean(o * o, axis=0, keepdims=True)
    out_ref[...] = jnp.maximum(
        g_ref[...] * o * lax.rsqrt(var + 1e-5) + be_ref[...], 0.0)


_layer_tc = pl.pallas_call(
    _layer_body,
    out_shape=jax.ShapeDtypeStruct((NN, DD), jnp.float32),
)


_BR = 2000


def _heads_body(h_ref, w1_ref, b1_ref, w2_ref, b2_ref,
                w30, w31, w32, w33, w34, b30, b31, b32, b33, b34,
                o0, o1, o2, o3, o4):
    hv = h_ref[...]
    w3s = (w30, w31, w32, w33, w34)
    b3s = (b30, b31, b32, b33, b34)
    outs = (o0, o1, o2, o3, o4)
    for i in range(5):
        z = jnp.maximum(
            jnp.dot(hv, w1_ref[i], preferred_element_type=jnp.float32)
            + b1_ref[i], 0.0)
        z = jnp.maximum(
            jnp.dot(z, w2_ref[i], preferred_element_type=jnp.float32)
            + b2_ref[i], 0.0)
        outs[i][...] = (jnp.dot(z, w3s[i][...],
                                preferred_element_type=jnp.float32)
                        + b3s[i][...])


_heads = pl.pallas_call(
    _heads_body,
    grid=(NN // _BR,),
    in_specs=[
        pl.BlockSpec((_BR, DD), lambda i: (i, 0)),
        pl.BlockSpec((5, DD, MM), lambda i: (0, 0, 0)),
        pl.BlockSpec((5, MM), lambda i: (0, 0)),
        pl.BlockSpec((5, MM, MM // 2), lambda i: (0, 0, 0)),
        pl.BlockSpec((5, MM // 2), lambda i: (0, 0)),
    ] + [pl.BlockSpec((MM // 2, o), lambda i: (0, 0)) for o in OUTS]
      + [pl.BlockSpec((1, o), lambda i: (0, 0)) for o in OUTS],
    out_specs=[pl.BlockSpec((_BR, o), lambda i: (i, 0)) for o in OUTS],
    out_shape=[jax.ShapeDtypeStruct((NN, o), jnp.float32) for o in OUTS],
)


def kernel(x, edge_index, Wl, Wr, bc, gamma, beta, alpha,
           W1, b1, W2, b2, W3, b3):
    src = edge_index[0]
    dst = edge_index[1]
    pad = R_PAD * CH - EE
    src_p = jnp.concatenate(
        [src, jnp.zeros((pad,), jnp.int32)]).reshape(R_PAD, CH)
    pad_dst = NN + (jnp.arange(pad, dtype=jnp.int32) % (N_PAD - NN))
    dst_p = jnp.concatenate([dst, pad_dst]).reshape(R_PAD, CH)

    zeros_deg = jnp.zeros((N_PAD, DEGW), jnp.float32)
    ones_deg = jnp.ones((CH, DEGW), jnp.float32)
    zeros_feat = jnp.zeros((N_PAD, DD), jnp.float32)

    deg_parts = _sc_degree(dst_p, zeros_deg, ones_deg)
    rdeg = _rdeg(deg_parts)

    h = x
    for i in range(5):
        s2 = _sc_segsum(h, src_p, dst_p, zeros_feat).reshape(NC, N_PAD, DD)
        h = _layer_tc(s2[0], s2[1], rdeg, h, Wl[i], Wr[i],
                      bc[i][None], gamma[i][None], beta[i][None],
                      alpha[i][None])

    outs = _heads(h, W1, b1, W2, b2, *W3, *[b[None] for b in b3])
    return tuple(outs)
